# baseline (speedup 1.0000x reference)
import functools

import jax
import jax.numpy as jnp
from jax import lax
from jax.experimental import pallas as pl
from jax.experimental.pallas import tpu as pltpu
from jax.experimental.pallas import tpu_sc as plsc

_VOCAB = 100000
_HIDDEN = 128
_BATCH = 4096
_TEXT_LEN = 200

_NUM_WORKERS = 32
_ROWS_PER_W = _BATCH // _NUM_WORKERS
_HB8 = _HIDDEN // 16
_LPAD = 256


def _sc_lookup_t(x_flat, W):
  mesh = plsc.VectorSubcoreMesh(core_axis_name="c", subcore_axis_name="s")

  @functools.partial(
      pl.kernel,
      out_type=jax.ShapeDtypeStruct((_BATCH, _HIDDEN * _TEXT_LEN),
                                    jnp.float32),
      mesh=mesh,
      compiler_params=pltpu.CompilerParams(needs_layout_passes=False),
      scratch_types=[
          pltpu.VMEM((2 * _LPAD,), jnp.int32),
          pltpu.VMEM((2, _LPAD, _HIDDEN), jnp.float32),
          pltpu.VMEM((2 * _HIDDEN * _TEXT_LEN,), jnp.float32),
          pltpu.SemaphoreType.DMA((2,)),
          pltpu.SemaphoreType.DMA((2,)),
      ],
  )
  def k(w_hbm, x_hbm, out_hbm, idx_v, g_v, t_v, gsems, osems):
    wid = lax.axis_index("s") * 2 + lax.axis_index("c")
    tok_base = wid * (_ROWS_PER_W * _TEXT_LEN)
    row_base = wid * _ROWS_PER_W

    def idx_view(slot):
      return idx_v.at[pl.ds(slot * _LPAD, _LPAD)]

    def gather(s, slot):
      pltpu.sync_copy(
          x_hbm.at[pl.ds(tok_base + s * _TEXT_LEN, _LPAD)],
          idx_view(slot))
      pltpu.async_copy(
          w_hbm.at[idx_view(slot)], g_v.at[slot], gsems.at[slot])

    def wait_gather(s, slot):
      pltpu.make_async_copy(
          w_hbm.at[idx_view(slot)], g_v.at[slot], gsems.at[slot]).wait()

    _TSZ = _HIDDEN * _TEXT_LEN

    def out_copy(s, slot):
      return pltpu.make_async_copy(
          t_v.at[pl.ds(slot * _TSZ, _TSZ)],
          out_hbm.at[row_base + s],
          osems.at[slot],
      )

    iotas = [(hb * 16 + lax.iota(jnp.int32, 16)) * _TEXT_LEN
             for hb in range(_HB8)]

    def transpose(slot):
      tbase = slot * (_HIDDEN * _TEXT_LEN)

      def lbody(l, carry):
        for hb in range(_HB8):
          v = g_v[slot, l, pl.ds(hb * 16, 16)]
          plsc.store_scatter(t_v, [iotas[hb] + (tbase + l)], v)
        return carry

      lax.fori_loop(0, _TEXT_LEN, lbody, 0)

    gather(0, 0)

    def step(s, carry):
      slot = lax.rem(s, 2)
      nslot = lax.rem(s + 1, 2)

      wait_gather(s, slot)

      @pl.when(s + 1 < _ROWS_PER_W)
      def _():
        gather(s + 1, nslot)

      @pl.when(s >= 2)
      def _():
        out_copy(s - 2, slot).wait()

      transpose(slot)
      out_copy(s, slot).start()
      return carry

    lax.fori_loop(0, _ROWS_PER_W, step, 0)

    out_copy(_ROWS_PER_W - 2, 0).wait()
    out_copy(_ROWS_PER_W - 1, 1).wait()

  return k(W, x_flat)


@jax.jit
def kernel(x, W):
  xf = x.astype(jnp.int32).reshape(-1)
  xf = jnp.concatenate([xf, jnp.zeros(_LPAD - _TEXT_LEN, jnp.int32)])
  out2d = _sc_lookup_t(xf, W)
  return out2d.reshape(_BATCH, _HIDDEN, _TEXT_LEN)


# speedup vs baseline: 1.0188x; 1.0188x over previous
import functools

import jax
import jax.numpy as jnp
from jax import lax
from jax.experimental import pallas as pl
from jax.experimental.pallas import tpu as pltpu
from jax.experimental.pallas import tpu_sc as plsc

_VOCAB = 100000
_HIDDEN = 128
_BATCH = 4096
_TEXT_LEN = 200

_NUM_WORKERS = 32
_ROWS_PER_W = _BATCH // _NUM_WORKERS
_HB8 = _HIDDEN // 16
_LPAD = 256


def _sc_lookup_t(x_flat, W):
  mesh = plsc.VectorSubcoreMesh(core_axis_name="c", subcore_axis_name="s")

  @functools.partial(
      pl.kernel,
      out_type=jax.ShapeDtypeStruct((_BATCH, _HIDDEN * _TEXT_LEN),
                                    jnp.float32),
      mesh=mesh,
      compiler_params=pltpu.CompilerParams(needs_layout_passes=False),
      scratch_types=[
          pltpu.VMEM((2 * _LPAD,), jnp.int32),
          pltpu.VMEM((2, _LPAD, _HIDDEN), jnp.float32),
          pltpu.VMEM((2 * _HIDDEN * _TEXT_LEN,), jnp.float32),
          pltpu.SemaphoreType.DMA((2,)),
          pltpu.SemaphoreType.DMA((2,)),
      ],
  )
  def k(w_hbm, x_hbm, out_hbm, idx_v, g_v, t_v, gsems, osems):
    wid = lax.axis_index("s") * 2 + lax.axis_index("c")
    tok_base = wid * (_ROWS_PER_W * _TEXT_LEN)
    row_base = wid * _ROWS_PER_W

    def idx_view(slot):
      return idx_v.at[pl.ds(slot * _LPAD, _LPAD)]

    def gather(s, slot):
      pltpu.sync_copy(
          x_hbm.at[pl.ds(tok_base + s * _TEXT_LEN, _LPAD)],
          idx_view(slot))
      pltpu.async_copy(
          w_hbm.at[idx_view(slot)], g_v.at[slot], gsems.at[slot])

    def wait_gather(s, slot):
      pltpu.make_async_copy(
          w_hbm.at[idx_view(slot)], g_v.at[slot], gsems.at[slot]).wait()

    _TSZ = _HIDDEN * _TEXT_LEN

    def out_copy(s, slot):
      return pltpu.make_async_copy(
          t_v.at[pl.ds(slot * _TSZ, _TSZ)],
          out_hbm.at[row_base + s],
          osems.at[slot],
      )

    iotas = [(hb * 16 + lax.iota(jnp.int32, 16)) * _TEXT_LEN
             for hb in range(_HB8)]

    def transpose(slot):
      tbase = slot * (_HIDDEN * _TEXT_LEN)

      def lbody(lq, carry):
        l0 = lq * 8
        for dl in range(8):
          l = l0 + dl
          for hb in range(_HB8):
            v = g_v[slot, l, pl.ds(hb * 16, 16)]
            plsc.store_scatter(t_v, [iotas[hb] + (tbase + l)], v)
        return carry

      lax.fori_loop(0, _TEXT_LEN // 8, lbody, 0)

    gather(0, 0)

    def step(s, carry):
      slot = lax.rem(s, 2)
      nslot = lax.rem(s + 1, 2)

      wait_gather(s, slot)

      @pl.when(s + 1 < _ROWS_PER_W)
      def _():
        gather(s + 1, nslot)

      @pl.when(s >= 2)
      def _():
        out_copy(s - 2, slot).wait()

      transpose(slot)
      out_copy(s, slot).start()
      return carry

    lax.fori_loop(0, _ROWS_PER_W, step, 0)

    out_copy(_ROWS_PER_W - 2, 0).wait()
    out_copy(_ROWS_PER_W - 1, 1).wait()

  return k(W, x_flat)


@jax.jit
def kernel(x, W):
  xf = x.astype(jnp.int32).reshape(-1)
  xf = jnp.concatenate([xf, jnp.zeros(_LPAD - _TEXT_LEN, jnp.int32)])
  out2d = _sc_lookup_t(xf, W)
  return out2d.reshape(_BATCH, _HIDDEN, _TEXT_LEN)


# speedup vs baseline: 1.4120x; 1.3860x over previous
import functools

import jax
import jax.numpy as jnp
from jax import lax
from jax.experimental import pallas as pl
from jax.experimental.pallas import tpu as pltpu
from jax.experimental.pallas import tpu_sc as plsc

_VOCAB = 100000
_HIDDEN = 128
_BATCH = 4096
_TEXT_LEN = 200

_NUM_WORKERS = 32
_IDX_ROWS = 4
_CHUNK = _IDX_ROWS * 128

_N_HALF = 2
_HB = _BATCH // _N_HALF
_BB = 64
_NB = _HB // _BB
_K = 4
_SB = _BB // _K


def _sc_gather(x_flat, W):
  n = x_flat.shape[0]
  per_w = n // _NUM_WORKERS
  steps = per_w // _CHUNK
  assert per_w % _CHUNK == 0

  mesh = plsc.VectorSubcoreMesh(core_axis_name="c", subcore_axis_name="s")

  @functools.partial(
      pl.kernel,
      out_type=jax.ShapeDtypeStruct((n, _HIDDEN), jnp.float32),
      mesh=mesh,
      scratch_types=[
          pltpu.VMEM((_CHUNK,), jnp.int32),
          pltpu.VMEM((_CHUNK, _HIDDEN), jnp.float32),
          pltpu.SemaphoreType.DMA,
      ],
  )
  def k(w_hbm, x_hbm, out_hbm, idx_v, rows_v, sem):
    wid = lax.axis_index("s") * 2 + lax.axis_index("c")
    base = wid * per_w

    def step(i, carry):
      off = base + i * _CHUNK
      pltpu.sync_copy(x_hbm.at[pl.ds(off, _CHUNK)], idx_v)
      copies = []
      for j in range(_IDX_ROWS):
        copies.append(
            pltpu.async_copy(
                w_hbm.at[idx_v.at[pl.ds(j * 128, 128)]],
                rows_v.at[pl.ds(j * 128, 128)],
                sem,
            ))
      for c in copies:
        c.wait()
      pltpu.sync_copy(rows_v, out_hbm.at[pl.ds(off, _CHUNK)])
      return carry

    lax.fori_loop(0, steps, step, 0)

  return k(W, x_flat)


def _make_transpose_body(base_block, n_extra_in):
  def body(*refs):
    g_ref, o_ref = refs[n_extra_in], refs[n_extra_in + 1]
    ibuf, obuf, isems, osems = refs[n_extra_in + 2:]
    i = pl.program_id(0)
    slot = lax.rem(i, 2)
    nslot = lax.rem(i + 1, 2)

    def in_copy(blk, s, k):
      return pltpu.make_async_copy(
          g_ref.at[pl.ds(blk * _BB + k * _SB, _SB)],
          ibuf.at[s, pl.ds(k * _SB, _SB)],
          isems.at[s, k],
      )

    def out_copy(blk, s, k):
      return pltpu.make_async_copy(
          obuf.at[s, pl.ds(k * _SB, _SB)],
          o_ref.at[pl.ds((base_block + blk) * _BB + k * _SB, _SB)],
          osems.at[s, k],
      )

    @pl.when(i == 0)
    def _():
      for k in range(_K):
        in_copy(0, 0, k).start()

    @pl.when(i >= 2)
    def _():
      for k in range(_K):
        out_copy(i - 2, slot, k).wait()

    @pl.when(i + 1 < _NB)
    def _():
      for k in range(_K):
        in_copy(i + 1, nslot, k).start()

    for k in range(_K):
      in_copy(i, slot, k).wait()

    obuf[slot] = jnp.transpose(ibuf[slot], (0, 2, 1))

    for k in range(_K):
      out_copy(i, slot, k).start()

    @pl.when(i == _NB - 1)
    def _():
      for k in range(_K):
        out_copy(_NB - 2, nslot, k).wait()
      for k in range(_K):
        out_copy(_NB - 1, slot, k).wait()

  return body


_SCRATCH = [
    pltpu.VMEM((2, _BB, _TEXT_LEN, _HIDDEN), jnp.float32),
    pltpu.VMEM((2, _BB, _HIDDEN, _TEXT_LEN), jnp.float32),
    pltpu.SemaphoreType.DMA((2, _K)),
    pltpu.SemaphoreType.DMA((2, _K)),
]

_OUT_SHAPE = jax.ShapeDtypeStruct((_BATCH, _HIDDEN, _TEXT_LEN), jnp.float32)


def _transpose_first(g):
  return pl.pallas_call(
      _make_transpose_body(0, 0),
      grid=(_NB,),
      in_specs=[pl.BlockSpec(memory_space=pl.ANY)],
      out_specs=pl.BlockSpec(memory_space=pl.ANY),
      out_shape=_OUT_SHAPE,
      scratch_shapes=_SCRATCH,
  )(g)


def _transpose_second(buf, g):
  return pl.pallas_call(
      _make_transpose_body(_NB, 1),
      grid=(_NB,),
      in_specs=[
          pl.BlockSpec(memory_space=pl.ANY),
          pl.BlockSpec(memory_space=pl.ANY),
      ],
      out_specs=pl.BlockSpec(memory_space=pl.ANY),
      out_shape=_OUT_SHAPE,
      input_output_aliases={0: 0},
      scratch_shapes=_SCRATCH,
  )(buf, g)


@jax.jit
def kernel(x, W):
  xi = x.astype(jnp.int32)
  x0 = xi[:_HB].reshape(-1)
  x1 = xi[_HB:].reshape(-1)
  g0 = _sc_gather(x0, W).reshape(_HB, _TEXT_LEN, _HIDDEN)
  g1 = _sc_gather(x1, W).reshape(_HB, _TEXT_LEN, _HIDDEN)
  buf = _transpose_first(g0)
  return _transpose_second(buf, g1)


# speedup vs baseline: 1.4366x; 1.0174x over previous
import functools

import jax
import jax.numpy as jnp
from jax import lax
from jax.experimental import pallas as pl
from jax.experimental.pallas import tpu as pltpu
from jax.experimental.pallas import tpu_sc as plsc

_VOCAB = 100000
_HIDDEN = 128
_BATCH = 4096
_TEXT_LEN = 200

_NUM_WORKERS = 32
_IDX_ROWS = 2
_CHUNK = _IDX_ROWS * 128

_N_HALF = 2
_HB = _BATCH // _N_HALF
_BB = 64
_NB = _HB // _BB
_K = 4
_SB = _BB // _K


def _sc_gather(x_flat, W):
  n = x_flat.shape[0]
  per_w = n // _NUM_WORKERS
  steps = per_w // _CHUNK
  assert per_w % _CHUNK == 0

  mesh = plsc.VectorSubcoreMesh(core_axis_name="c", subcore_axis_name="s")

  @functools.partial(
      pl.kernel,
      out_type=jax.ShapeDtypeStruct((n, _HIDDEN), jnp.float32),
      mesh=mesh,
      scratch_types=[
          pltpu.VMEM((2, _CHUNK), jnp.int32),
          pltpu.VMEM((2, _CHUNK, _HIDDEN), jnp.float32),
          pltpu.SemaphoreType.DMA((2,)),
          pltpu.SemaphoreType.DMA((2,)),
      ],
  )
  def k(w_hbm, x_hbm, out_hbm, idx_v, rows_v, gsems, osems):
    wid = lax.axis_index("s") * 2 + lax.axis_index("c")
    base = wid * per_w

    def gathers(i, slot):
      off = base + i * _CHUNK
      pltpu.sync_copy(x_hbm.at[pl.ds(off, _CHUNK)], idx_v.at[slot])
      for j in range(_IDX_ROWS):
        pltpu.async_copy(
            w_hbm.at[idx_v.at[slot, pl.ds(j * 128, 128)]],
            rows_v.at[slot, pl.ds(j * 128, 128)],
            gsems.at[slot],
        )

    def wait_gathers(i, slot):
      for j in range(_IDX_ROWS):
        pltpu.make_async_copy(
            w_hbm.at[idx_v.at[slot, pl.ds(j * 128, 128)]],
            rows_v.at[slot, pl.ds(j * 128, 128)],
            gsems.at[slot],
        ).wait()

    def out_copy(i, slot):
      return pltpu.make_async_copy(
          rows_v.at[slot],
          out_hbm.at[pl.ds(base + i * _CHUNK, _CHUNK)],
          osems.at[slot],
      )

    gathers(0, 0)

    def step(i, carry):
      slot = lax.rem(i, 2)
      nslot = lax.rem(i + 1, 2)

      @pl.when(i + 1 < steps)
      def _():
        @pl.when(i >= 1)
        def _():
          out_copy(i - 1, nslot).wait()

        gathers(i + 1, nslot)

      wait_gathers(i, slot)
      out_copy(i, slot).start()
      return carry

    lax.fori_loop(0, steps, step, 0)

    out_copy(steps - 2, lax.rem(steps - 2, 2)).wait()
    out_copy(steps - 1, lax.rem(steps - 1, 2)).wait()

  return k(W, x_flat)


def _make_transpose_body(base_block, n_extra_in):
  def body(*refs):
    g_ref, o_ref = refs[n_extra_in], refs[n_extra_in + 1]
    ibuf, obuf, isems, osems = refs[n_extra_in + 2:]
    i = pl.program_id(0)
    slot = lax.rem(i, 2)
    nslot = lax.rem(i + 1, 2)

    def in_copy(blk, s, k):
      return pltpu.make_async_copy(
          g_ref.at[pl.ds(blk * _BB + k * _SB, _SB)],
          ibuf.at[s, pl.ds(k * _SB, _SB)],
          isems.at[s, k],
      )

    def out_copy(blk, s, k):
      return pltpu.make_async_copy(
          obuf.at[s, pl.ds(k * _SB, _SB)],
          o_ref.at[pl.ds((base_block + blk) * _BB + k * _SB, _SB)],
          osems.at[s, k],
      )

    @pl.when(i == 0)
    def _():
      for k in range(_K):
        in_copy(0, 0, k).start()

    @pl.when(i >= 2)
    def _():
      for k in range(_K):
        out_copy(i - 2, slot, k).wait()

    @pl.when(i + 1 < _NB)
    def _():
      for k in range(_K):
        in_copy(i + 1, nslot, k).start()

    for k in range(_K):
      in_copy(i, slot, k).wait()

    obuf[slot] = jnp.transpose(ibuf[slot], (0, 2, 1))

    for k in range(_K):
      out_copy(i, slot, k).start()

    @pl.when(i == _NB - 1)
    def _():
      for k in range(_K):
        out_copy(_NB - 2, nslot, k).wait()
      for k in range(_K):
        out_copy(_NB - 1, slot, k).wait()

  return body


_SCRATCH = [
    pltpu.VMEM((2, _BB, _TEXT_LEN, _HIDDEN), jnp.float32),
    pltpu.VMEM((2, _BB, _HIDDEN, _TEXT_LEN), jnp.float32),
    pltpu.SemaphoreType.DMA((2, _K)),
    pltpu.SemaphoreType.DMA((2, _K)),
]

_OUT_SHAPE = jax.ShapeDtypeStruct((_BATCH, _HIDDEN, _TEXT_LEN), jnp.float32)


def _transpose_first(g):
  return pl.pallas_call(
      _make_transpose_body(0, 0),
      grid=(_NB,),
      in_specs=[pl.BlockSpec(memory_space=pl.ANY)],
      out_specs=pl.BlockSpec(memory_space=pl.ANY),
      out_shape=_OUT_SHAPE,
      scratch_shapes=_SCRATCH,
  )(g)


def _transpose_second(buf, g):
  return pl.pallas_call(
      _make_transpose_body(_NB, 1),
      grid=(_NB,),
      in_specs=[
          pl.BlockSpec(memory_space=pl.ANY),
          pl.BlockSpec(memory_space=pl.ANY),
      ],
      out_specs=pl.BlockSpec(memory_space=pl.ANY),
      out_shape=_OUT_SHAPE,
      input_output_aliases={0: 0},
      scratch_shapes=_SCRATCH,
  )(buf, g)


@jax.jit
def kernel(x, W):
  xi = x.astype(jnp.int32)
  x0 = xi[:_HB].reshape(-1)
  x1 = xi[_HB:].reshape(-1)
  g0 = _sc_gather(x0, W).reshape(_HB, _TEXT_LEN, _HIDDEN)
  g1 = _sc_gather(x1, W).reshape(_HB, _TEXT_LEN, _HIDDEN)
  buf = _transpose_first(g0)
  return _transpose_second(buf, g1)


# speedup vs baseline: 1.4414x; 1.0033x over previous
import functools

import jax
import jax.numpy as jnp
from jax import lax
from jax.experimental import pallas as pl
from jax.experimental.pallas import tpu as pltpu
from jax.experimental.pallas import tpu_sc as plsc

_VOCAB = 100000
_HIDDEN = 128
_BATCH = 4096
_TEXT_LEN = 200

_NUM_WORKERS = 32
_IDX_ROWS = 2
_CHUNK = _IDX_ROWS * 128

_N_PART = 4
_HB = _BATCH // _N_PART
_BB = 64
_NB = _HB // _BB
_K = 4
_SB = _BB // _K


def _sc_gather(x_flat, W):
  n = x_flat.shape[0]
  per_w = n // _NUM_WORKERS
  steps = per_w // _CHUNK
  assert per_w % _CHUNK == 0

  mesh = plsc.VectorSubcoreMesh(core_axis_name="c", subcore_axis_name="s")

  @functools.partial(
      pl.kernel,
      out_type=jax.ShapeDtypeStruct((n, _HIDDEN), jnp.float32),
      mesh=mesh,
      scratch_types=[
          pltpu.VMEM((2, _CHUNK), jnp.int32),
          pltpu.VMEM((2, _CHUNK, _HIDDEN), jnp.float32),
          pltpu.SemaphoreType.DMA((2,)),
          pltpu.SemaphoreType.DMA((2,)),
      ],
  )
  def k(w_hbm, x_hbm, out_hbm, idx_v, rows_v, gsems, osems):
    wid = lax.axis_index("s") * 2 + lax.axis_index("c")
    base = wid * per_w

    def gathers(i, slot):
      off = base + i * _CHUNK
      pltpu.sync_copy(x_hbm.at[pl.ds(off, _CHUNK)], idx_v.at[slot])
      for j in range(_IDX_ROWS):
        pltpu.async_copy(
            w_hbm.at[idx_v.at[slot, pl.ds(j * 128, 128)]],
            rows_v.at[slot, pl.ds(j * 128, 128)],
            gsems.at[slot],
        )

    def wait_gathers(i, slot):
      for j in range(_IDX_ROWS):
        pltpu.make_async_copy(
            w_hbm.at[idx_v.at[slot, pl.ds(j * 128, 128)]],
            rows_v.at[slot, pl.ds(j * 128, 128)],
            gsems.at[slot],
        ).wait()

    def out_copy(i, slot):
      return pltpu.make_async_copy(
          rows_v.at[slot],
          out_hbm.at[pl.ds(base + i * _CHUNK, _CHUNK)],
          osems.at[slot],
      )

    gathers(0, 0)

    def step(i, carry):
      slot = lax.rem(i, 2)
      nslot = lax.rem(i + 1, 2)

      @pl.when(i + 1 < steps)
      def _():
        @pl.when(i >= 1)
        def _():
          out_copy(i - 1, nslot).wait()

        gathers(i + 1, nslot)

      wait_gathers(i, slot)
      out_copy(i, slot).start()
      return carry

    lax.fori_loop(0, steps, step, 0)

    out_copy(steps - 2, lax.rem(steps - 2, 2)).wait()
    out_copy(steps - 1, lax.rem(steps - 1, 2)).wait()

  return k(W, x_flat)


def _make_transpose_body(base_block, n_extra_in):
  def body(*refs):
    g_ref, o_ref = refs[n_extra_in], refs[n_extra_in + 1]
    ibuf, obuf, isems, osems = refs[n_extra_in + 2:]
    i = pl.program_id(0)
    slot = lax.rem(i, 2)
    nslot = lax.rem(i + 1, 2)

    def in_copy(blk, s, k):
      return pltpu.make_async_copy(
          g_ref.at[pl.ds(blk * _BB + k * _SB, _SB)],
          ibuf.at[s, pl.ds(k * _SB, _SB)],
          isems.at[s, k],
      )

    def out_copy(blk, s, k):
      return pltpu.make_async_copy(
          obuf.at[s, pl.ds(k * _SB, _SB)],
          o_ref.at[pl.ds((base_block + blk) * _BB + k * _SB, _SB)],
          osems.at[s, k],
      )

    @pl.when(i == 0)
    def _():
      for k in range(_K):
        in_copy(0, 0, k).start()

    @pl.when(i >= 2)
    def _():
      for k in range(_K):
        out_copy(i - 2, slot, k).wait()

    @pl.when(i + 1 < _NB)
    def _():
      for k in range(_K):
        in_copy(i + 1, nslot, k).start()

    for k in range(_K):
      in_copy(i, slot, k).wait()

    obuf[slot] = jnp.transpose(ibuf[slot], (0, 2, 1))

    for k in range(_K):
      out_copy(i, slot, k).start()

    @pl.when(i == _NB - 1)
    def _():
      for k in range(_K):
        out_copy(_NB - 2, nslot, k).wait()
      for k in range(_K):
        out_copy(_NB - 1, slot, k).wait()

  return body


_SCRATCH = [
    pltpu.VMEM((2, _BB, _TEXT_LEN, _HIDDEN), jnp.float32),
    pltpu.VMEM((2, _BB, _HIDDEN, _TEXT_LEN), jnp.float32),
    pltpu.SemaphoreType.DMA((2, _K)),
    pltpu.SemaphoreType.DMA((2, _K)),
]

_OUT_SHAPE = jax.ShapeDtypeStruct((_BATCH, _HIDDEN, _TEXT_LEN), jnp.float32)


def _transpose_first(g):
  return pl.pallas_call(
      _make_transpose_body(0, 0),
      grid=(_NB,),
      in_specs=[pl.BlockSpec(memory_space=pl.ANY)],
      out_specs=pl.BlockSpec(memory_space=pl.ANY),
      out_shape=_OUT_SHAPE,
      scratch_shapes=_SCRATCH,
  )(g)


def _transpose_next(buf, g, q):
  return pl.pallas_call(
      _make_transpose_body(q * _NB, 1),
      grid=(_NB,),
      in_specs=[
          pl.BlockSpec(memory_space=pl.ANY),
          pl.BlockSpec(memory_space=pl.ANY),
      ],
      out_specs=pl.BlockSpec(memory_space=pl.ANY),
      out_shape=_OUT_SHAPE,
      input_output_aliases={0: 0},
      scratch_shapes=_SCRATCH,
  )(buf, g)


@jax.jit
def kernel(x, W):
  xi = x.astype(jnp.int32)
  gs = [
      _sc_gather(xi[q * _HB:(q + 1) * _HB].reshape(-1), W)
      .reshape(_HB, _TEXT_LEN, _HIDDEN)
      for q in range(_N_PART)
  ]
  buf = _transpose_first(gs[0])
  for q in range(1, _N_PART):
    buf = _transpose_next(buf, gs[q], q)
  return buf


# speedup vs baseline: 1.4428x; 1.0010x over previous
import functools

import jax
import jax.numpy as jnp
from jax import lax
from jax.experimental import pallas as pl
from jax.experimental.pallas import tpu as pltpu
from jax.experimental.pallas import tpu_sc as plsc

_VOCAB = 100000
_HIDDEN = 128
_BATCH = 4096
_TEXT_LEN = 200

_NUM_WORKERS = 32
_IDX_ROWS = 2
_CHUNK = _IDX_ROWS * 128

_N_PART = 4
_HB = _BATCH // _N_PART
_BB = 64
_NB = _HB // _BB
_K = 8
_SB = _BB // _K


def _sc_gather(x_flat, W):
  n = x_flat.shape[0]
  per_w = n // _NUM_WORKERS
  steps = per_w // _CHUNK
  assert per_w % _CHUNK == 0

  mesh = plsc.VectorSubcoreMesh(core_axis_name="c", subcore_axis_name="s")

  @functools.partial(
      pl.kernel,
      out_type=jax.ShapeDtypeStruct((n, _HIDDEN), jnp.float32),
      mesh=mesh,
      scratch_types=[
          pltpu.VMEM((2, _CHUNK), jnp.int32),
          pltpu.VMEM((2, _CHUNK, _HIDDEN), jnp.float32),
          pltpu.SemaphoreType.DMA((2,)),
          pltpu.SemaphoreType.DMA((2,)),
      ],
  )
  def k(w_hbm, x_hbm, out_hbm, idx_v, rows_v, gsems, osems):
    wid = lax.axis_index("s") * 2 + lax.axis_index("c")
    base = wid * per_w

    def gathers(i, slot):
      off = base + i * _CHUNK
      pltpu.sync_copy(x_hbm.at[pl.ds(off, _CHUNK)], idx_v.at[slot])
      for j in range(_IDX_ROWS):
        pltpu.async_copy(
            w_hbm.at[idx_v.at[slot, pl.ds(j * 128, 128)]],
            rows_v.at[slot, pl.ds(j * 128, 128)],
            gsems.at[slot],
        )

    def wait_gathers(i, slot):
      for j in range(_IDX_ROWS):
        pltpu.make_async_copy(
            w_hbm.at[idx_v.at[slot, pl.ds(j * 128, 128)]],
            rows_v.at[slot, pl.ds(j * 128, 128)],
            gsems.at[slot],
        ).wait()

    def out_copy(i, slot):
      return pltpu.make_async_copy(
          rows_v.at[slot],
          out_hbm.at[pl.ds(base + i * _CHUNK, _CHUNK)],
          osems.at[slot],
      )

    gathers(0, 0)

    def step(i, carry):
      slot = lax.rem(i, 2)
      nslot = lax.rem(i + 1, 2)

      @pl.when(i + 1 < steps)
      def _():
        @pl.when(i >= 1)
        def _():
          out_copy(i - 1, nslot).wait()

        gathers(i + 1, nslot)

      wait_gathers(i, slot)
      out_copy(i, slot).start()
      return carry

    lax.fori_loop(0, steps, step, 0)

    out_copy(steps - 2, lax.rem(steps - 2, 2)).wait()
    out_copy(steps - 1, lax.rem(steps - 1, 2)).wait()

  return k(W, x_flat)


def _make_transpose_body(base_block, n_extra_in):
  def body(*refs):
    g_ref, o_ref = refs[n_extra_in], refs[n_extra_in + 1]
    ibuf, obuf, isems, osems = refs[n_extra_in + 2:]
    i = pl.program_id(0)
    slot = lax.rem(i, 2)
    nslot = lax.rem(i + 1, 2)

    def in_copy(blk, s, k):
      return pltpu.make_async_copy(
          g_ref.at[pl.ds(blk * _BB + k * _SB, _SB)],
          ibuf.at[s, pl.ds(k * _SB, _SB)],
          isems.at[s, k],
      )

    def out_copy(blk, s, k):
      return pltpu.make_async_copy(
          obuf.at[s, pl.ds(k * _SB, _SB)],
          o_ref.at[pl.ds((base_block + blk) * _BB + k * _SB, _SB)],
          osems.at[s, k],
      )

    @pl.when(i == 0)
    def _():
      for k in range(_K):
        in_copy(0, 0, k).start()

    @pl.when(i >= 2)
    def _():
      for k in range(_K):
        out_copy(i - 2, slot, k).wait()

    @pl.when(i + 1 < _NB)
    def _():
      for k in range(_K):
        in_copy(i + 1, nslot, k).start()

    for k in range(_K):
      in_copy(i, slot, k).wait()

    obuf[slot] = jnp.transpose(ibuf[slot], (0, 2, 1))

    for k in range(_K):
      out_copy(i, slot, k).start()

    @pl.when(i == _NB - 1)
    def _():
      for k in range(_K):
        out_copy(_NB - 2, nslot, k).wait()
      for k in range(_K):
        out_copy(_NB - 1, slot, k).wait()

  return body


_SCRATCH = [
    pltpu.VMEM((2, _BB, _TEXT_LEN, _HIDDEN), jnp.float32),
    pltpu.VMEM((2, _BB, _HIDDEN, _TEXT_LEN), jnp.float32),
    pltpu.SemaphoreType.DMA((2, _K)),
    pltpu.SemaphoreType.DMA((2, _K)),
]

_OUT_SHAPE = jax.ShapeDtypeStruct((_BATCH, _HIDDEN, _TEXT_LEN), jnp.float32)


def _transpose_first(g):
  return pl.pallas_call(
      _make_transpose_body(0, 0),
      grid=(_NB,),
      in_specs=[pl.BlockSpec(memory_space=pl.ANY)],
      out_specs=pl.BlockSpec(memory_space=pl.ANY),
      out_shape=_OUT_SHAPE,
      scratch_shapes=_SCRATCH,
  )(g)


def _transpose_next(buf, g, q):
  return pl.pallas_call(
      _make_transpose_body(q * _NB, 1),
      grid=(_NB,),
      in_specs=[
          pl.BlockSpec(memory_space=pl.ANY),
          pl.BlockSpec(memory_space=pl.ANY),
      ],
      out_specs=pl.BlockSpec(memory_space=pl.ANY),
      out_shape=_OUT_SHAPE,
      input_output_aliases={0: 0},
      scratch_shapes=_SCRATCH,
  )(buf, g)


@jax.jit
def kernel(x, W):
  xi = x.astype(jnp.int32)
  gs = [
      _sc_gather(xi[q * _HB:(q + 1) * _HB].reshape(-1), W)
      .reshape(_HB, _TEXT_LEN, _HIDDEN)
      for q in range(_N_PART)
  ]
  buf = _transpose_first(gs[0])
  for q in range(1, _N_PART):
    buf = _transpose_next(buf, gs[q], q)
  return buf
